# baseline (speedup 1.0000x reference)
import functools

import jax
import jax.numpy as jnp
from jax import lax
from jax.experimental import pallas as pl
from jax.experimental.pallas import tpu as pltpu

_IOU_T = 0.5
_BLK = 1280


def _nms_body(bt_ref, brow_ref, st_ref, ot_ref, os_ref,
              s_ref, keep_ref, *, n_pad, blk):
    i = pl.program_id(0)
    j = pl.program_id(1)
    ibase = pl.multiple_of(i * blk, blk)
    jbase = pl.multiple_of(j * blk, blk)

    @pl.when((i == 0) & (j == 0))
    def _():
        keep_ref[...] = jnp.ones_like(keep_ref)

    @pl.when(j >= i)
    def _active():
        x1 = brow_ref[:, 0:1]
        y1 = brow_ref[:, 1:2]
        x2 = brow_ref[:, 2:3]
        y2 = brow_ref[:, 3:4]
        cx1 = bt_ref[0:1, pl.ds(jbase, blk)]
        cy1 = bt_ref[1:2, pl.ds(jbase, blk)]
        cx2 = bt_ref[2:3, pl.ds(jbase, blk)]
        cy2 = bt_ref[3:4, pl.ds(jbase, blk)]
        area_r = (x2 - x1) * (y2 - y1)
        area_c = (cx2 - cx1) * (cy2 - cy1)
        iw = jnp.maximum(jnp.minimum(x2, cx2) - jnp.maximum(x1, cx1), 0.0)
        ih = jnp.maximum(jnp.minimum(y2, cy2) - jnp.maximum(y1, cy1), 0.0)
        inter = iw * ih
        union = jnp.maximum(area_r + area_c - inter, 1e-9)
        s = jnp.where(inter > _IOU_T * union, 1.0, 0.0)

        @pl.when(j == i)
        def _diagonal():
            tri_r = lax.broadcasted_iota(jnp.int32, (blk, blk), 0)
            tri_c = lax.broadcasted_iota(jnp.int32, (blk, blk), 1)
            s_ref[...] = jnp.where(tri_c > tri_r, s, 0.0)

            keep0 = keep_ref[:, pl.ds(ibase, blk)]

            def _cond(carry):
                return carry[1]

            def _sweep(carry):
                kb, _ = carry
                cnt = jnp.dot(kb, s_ref[...],
                              preferred_element_type=jnp.float32)
                kb_new = keep0 * jnp.where(cnt > 0.5, 0.0, 1.0)
                changed = jnp.sum(jnp.abs(kb_new - kb)) > 0.0
                return (kb_new, changed)

            kb, _ = lax.while_loop(_cond, _sweep, (keep0, True))
            keep_ref[:, pl.ds(ibase, blk)] = kb

            ot_ref[...] = bt_ref[:, pl.ds(ibase, blk)] * kb
            os_ref[...] = st_ref[:, pl.ds(ibase, blk)] * kb

        @pl.when(j > i)
        def _tail():
            kb = keep_ref[:, pl.ds(ibase, blk)]
            cnt = jnp.dot(kb, s, preferred_element_type=jnp.float32)
            keep_ref[:, pl.ds(jbase, blk)] = (
                keep_ref[:, pl.ds(jbase, blk)]
                * jnp.where(cnt > 0.5, 0.0, 1.0))


def kernel(boxes, scores):
    n = boxes.shape[0]
    blk = _BLK
    nb = -(-n // blk)
    n_pad = nb * blk

    order = jnp.argsort(-scores)
    b = jnp.take(boxes, order, axis=0)
    s = jnp.take(scores, order, axis=0)
    bp = jnp.concatenate(
        [b, jnp.zeros((n_pad - n, 4), jnp.float32)], axis=0)
    st = jnp.concatenate(
        [s, jnp.zeros((n_pad - n,), jnp.float32)], axis=0)[None, :]
    bt = bp.T

    ot, ost = pl.pallas_call(
        functools.partial(_nms_body, n_pad=n_pad, blk=blk),
        grid=(nb, nb),
        in_specs=[
            pl.BlockSpec((4, n_pad), lambda i, j: (0, 0)),
            pl.BlockSpec((blk, 4), lambda i, j: (i, 0)),
            pl.BlockSpec((1, n_pad), lambda i, j: (0, 0)),
        ],
        out_specs=[
            pl.BlockSpec((4, blk), lambda i, j: (0, i)),
            pl.BlockSpec((1, blk), lambda i, j: (0, i)),
        ],
        out_shape=[
            jax.ShapeDtypeStruct((4, n_pad), jnp.float32),
            jax.ShapeDtypeStruct((1, n_pad), jnp.float32),
        ],
        scratch_shapes=[
            pltpu.VMEM((blk, blk), jnp.float32),
            pltpu.VMEM((1, n_pad), jnp.float32),
        ],
    )(bt, bp, st)

    return jnp.concatenate([ot, ost], axis=0).T[:n]


# speedup vs baseline: 1.0017x; 1.0017x over previous
import functools

import jax
import jax.numpy as jnp
from jax import lax
from jax.experimental import pallas as pl
from jax.experimental.pallas import tpu as pltpu

_IOU_T = 0.5
_BLK = 1024


def _nms_body(bt_ref, brow_ref, st_ref, ot_ref, os_ref,
              s_ref, keep_ref, *, n_pad, blk):
    i = pl.program_id(0)
    j = pl.program_id(1)
    ibase = pl.multiple_of(i * blk, blk)
    jbase = pl.multiple_of(j * blk, blk)

    @pl.when((i == 0) & (j == 0))
    def _():
        keep_ref[...] = jnp.ones_like(keep_ref)

    @pl.when(j >= i)
    def _active():
        x1 = brow_ref[:, 0:1]
        y1 = brow_ref[:, 1:2]
        x2 = brow_ref[:, 2:3]
        y2 = brow_ref[:, 3:4]
        cx1 = bt_ref[0:1, pl.ds(jbase, blk)]
        cy1 = bt_ref[1:2, pl.ds(jbase, blk)]
        cx2 = bt_ref[2:3, pl.ds(jbase, blk)]
        cy2 = bt_ref[3:4, pl.ds(jbase, blk)]
        area_r = (x2 - x1) * (y2 - y1)
        area_c = (cx2 - cx1) * (cy2 - cy1)
        iw = jnp.maximum(jnp.minimum(x2, cx2) - jnp.maximum(x1, cx1), 0.0)
        ih = jnp.maximum(jnp.minimum(y2, cy2) - jnp.maximum(y1, cy1), 0.0)
        inter = iw * ih
        union = jnp.maximum(area_r + area_c - inter, 1e-9)
        s = jnp.where(inter > _IOU_T * union, 1.0, 0.0)

        @pl.when(j == i)
        def _diagonal():
            tri_r = lax.broadcasted_iota(jnp.int32, (blk, blk), 0)
            tri_c = lax.broadcasted_iota(jnp.int32, (blk, blk), 1)
            s_ref[...] = jnp.where(tri_c > tri_r, s, 0.0)

            keep0 = keep_ref[:, pl.ds(ibase, blk)]

            def _cond(carry):
                return carry[1]

            def _sweep(carry):
                kb, _ = carry
                cnt = jnp.dot(kb, s_ref[...],
                              preferred_element_type=jnp.float32)
                kb_new = keep0 * jnp.where(cnt > 0.5, 0.0, 1.0)
                changed = jnp.sum(jnp.abs(kb_new - kb)) > 0.0
                return (kb_new, changed)

            kb, _ = lax.while_loop(_cond, _sweep, (keep0, True))
            keep_ref[:, pl.ds(ibase, blk)] = kb

            ot_ref[...] = bt_ref[:, pl.ds(ibase, blk)] * kb
            os_ref[...] = st_ref[:, pl.ds(ibase, blk)] * kb

        @pl.when(j > i)
        def _tail():
            kb = keep_ref[:, pl.ds(ibase, blk)]
            cnt = jnp.dot(kb, s, preferred_element_type=jnp.float32)
            keep_ref[:, pl.ds(jbase, blk)] = (
                keep_ref[:, pl.ds(jbase, blk)]
                * jnp.where(cnt > 0.5, 0.0, 1.0))


def kernel(boxes, scores):
    n = boxes.shape[0]
    blk = _BLK
    nb = -(-n // blk)
    n_pad = nb * blk

    order = jnp.argsort(-scores)
    b = jnp.take(boxes, order, axis=0)
    s = jnp.take(scores, order, axis=0)
    bp = jnp.concatenate(
        [b, jnp.zeros((n_pad - n, 4), jnp.float32)], axis=0)
    st = jnp.concatenate(
        [s, jnp.zeros((n_pad - n,), jnp.float32)], axis=0)[None, :]
    bt = bp.T

    ot, ost = pl.pallas_call(
        functools.partial(_nms_body, n_pad=n_pad, blk=blk),
        grid=(nb, nb),
        in_specs=[
            pl.BlockSpec((4, n_pad), lambda i, j: (0, 0)),
            pl.BlockSpec((blk, 4), lambda i, j: (i, 0)),
            pl.BlockSpec((1, n_pad), lambda i, j: (0, 0)),
        ],
        out_specs=[
            pl.BlockSpec((4, blk), lambda i, j: (0, i)),
            pl.BlockSpec((1, blk), lambda i, j: (0, i)),
        ],
        out_shape=[
            jax.ShapeDtypeStruct((4, n_pad), jnp.float32),
            jax.ShapeDtypeStruct((1, n_pad), jnp.float32),
        ],
        scratch_shapes=[
            pltpu.VMEM((blk, blk), jnp.float32),
            pltpu.VMEM((1, n_pad), jnp.float32),
        ],
    )(bt, bp, st)

    return jnp.concatenate([ot, ost], axis=0).T[:n]


# speedup vs baseline: 1.0825x; 1.0807x over previous
import functools

import jax
import jax.numpy as jnp
from jax import lax
from jax.experimental import pallas as pl
from jax.experimental.pallas import tpu as pltpu

_BLK = 1024


def _nms_body(bt_ref, brow_ref, ob_ref, os_ref,
              s_ref, keep_ref, kbc_ref, *, blk):
    i = pl.program_id(0)
    j = pl.program_id(1)
    ibase = pl.multiple_of(i * blk, blk)
    jbase = pl.multiple_of(j * blk, blk)

    @pl.when((i == 0) & (j == 0))
    def _():
        keep_ref[...] = jnp.ones_like(keep_ref)

    @pl.when(j == i)
    def _diagonal():
        x1 = brow_ref[:, 0:1]
        y1 = brow_ref[:, 1:2]
        x2 = brow_ref[:, 2:3]
        y2 = brow_ref[:, 3:4]
        ar = brow_ref[:, 4:5]
        cx1 = bt_ref[0:1, pl.ds(ibase, blk)]
        cy1 = bt_ref[1:2, pl.ds(ibase, blk)]
        cx2 = bt_ref[2:3, pl.ds(ibase, blk)]
        cy2 = bt_ref[3:4, pl.ds(ibase, blk)]
        ac = bt_ref[4:5, pl.ds(ibase, blk)]
        iw = jnp.maximum(jnp.minimum(x2, cx2) - jnp.maximum(x1, cx1), 0.0)
        ih = jnp.minimum(y2, cy2) - jnp.maximum(y1, cy1)
        overlap = 3.0 * (iw * ih) > ar + ac
        tri_r = lax.broadcasted_iota(jnp.int32, (blk, blk), 0)
        tri_c = lax.broadcasted_iota(jnp.int32, (blk, blk), 1)
        s_ref[...] = jnp.where((tri_c > tri_r) & overlap, 1.0, 0.0)

        keep0 = keep_ref[:, pl.ds(ibase, blk)]

        def _step(kb):
            cnt = jnp.dot(kb, s_ref[...],
                          preferred_element_type=jnp.float32)
            return keep0 * jnp.where(cnt > 0.5, 0.0, 1.0)

        def _cond(carry):
            return carry[1]

        def _sweep(carry):
            kb, _ = carry
            kb1 = _step(kb)
            kb2 = _step(kb1)
            changed = jnp.sum(jnp.abs(kb2 - kb1)) > 0.0
            return (kb2, changed)

        kb, _ = lax.while_loop(_cond, _sweep, (keep0, True))
        keep_ref[:, pl.ds(ibase, blk)] = kb
        kbc = kb.T
        kbc_ref[...] = kbc

        ob_ref[...] = brow_ref[:, 0:4] * kbc
        os_ref[...] = brow_ref[:, 5:6] * kbc

    @pl.when(j > i)
    def _tail():
        kbc = kbc_ref[...]
        x1 = jnp.where(kbc > 0.0, brow_ref[:, 0:1], 3.0e4)
        y1 = brow_ref[:, 1:2]
        x2 = brow_ref[:, 2:3]
        y2 = brow_ref[:, 3:4]
        ar = brow_ref[:, 4:5]
        cx1 = bt_ref[0:1, pl.ds(jbase, blk)]
        cy1 = bt_ref[1:2, pl.ds(jbase, blk)]
        cx2 = bt_ref[2:3, pl.ds(jbase, blk)]
        cy2 = bt_ref[3:4, pl.ds(jbase, blk)]
        ac = bt_ref[4:5, pl.ds(jbase, blk)]
        iw = jnp.maximum(jnp.minimum(x2, cx2) - jnp.maximum(x1, cx1), 0.0)
        ih = jnp.minimum(y2, cy2) - jnp.maximum(y1, cy1)
        overlap = 3.0 * (iw * ih) > ar + ac
        sup = jnp.any(overlap, axis=0, keepdims=True)
        keep_ref[:, pl.ds(jbase, blk)] = jnp.where(
            sup, 0.0, keep_ref[:, pl.ds(jbase, blk)])


def kernel(boxes, scores):
    n = boxes.shape[0]
    blk = _BLK
    nb = -(-n // blk)
    n_pad = nb * blk

    order = jnp.argsort(-scores)
    b = jnp.take(boxes, order, axis=0)
    s = jnp.take(scores, order, axis=0)
    area = ((b[:, 2] - b[:, 0]) * (b[:, 3] - b[:, 1]))[:, None]
    rows = jnp.concatenate([b, area, s[:, None]], axis=1)
    rows = jnp.concatenate(
        [rows, jnp.zeros((n_pad - n, 6), jnp.float32)], axis=0)
    bt = rows[:, 0:5].T

    ob, os = pl.pallas_call(
        functools.partial(_nms_body, blk=blk),
        grid=(nb, nb),
        in_specs=[
            pl.BlockSpec((5, n_pad), lambda i, j: (0, 0)),
            pl.BlockSpec((blk, 6), lambda i, j: (i, 0)),
        ],
        out_specs=[
            pl.BlockSpec((blk, 4), lambda i, j: (i, 0)),
            pl.BlockSpec((blk, 1), lambda i, j: (i, 0)),
        ],
        out_shape=[
            jax.ShapeDtypeStruct((n_pad, 4), jnp.float32),
            jax.ShapeDtypeStruct((n_pad, 1), jnp.float32),
        ],
        scratch_shapes=[
            pltpu.VMEM((blk, blk), jnp.float32),
            pltpu.VMEM((1, n_pad), jnp.float32),
            pltpu.VMEM((blk, 1), jnp.float32),
        ],
    )(bt, rows)

    return jnp.concatenate([ob, os], axis=1)[:n]


# speedup vs baseline: 1.8868x; 1.7429x over previous
import functools

import jax
import jax.numpy as jnp
from jax import lax
from jax.experimental import pallas as pl
from jax.experimental.pallas import tpu as pltpu

_BLK = 1024


def _nms_body(bt_ref, brow_ref, out_ref,
              s_ref, keep_ref, kbc_ref, *, blk):
    i = pl.program_id(0)
    j = pl.program_id(1)
    ibase = pl.multiple_of(i * blk, blk)
    jbase = pl.multiple_of(j * blk, blk)

    @pl.when((i == 0) & (j == 0))
    def _():
        keep_ref[...] = jnp.ones_like(keep_ref)

    @pl.when(j == i)
    def _diagonal():
        x1 = brow_ref[:, 0:1]
        y1 = brow_ref[:, 1:2]
        x2 = brow_ref[:, 2:3]
        y2 = brow_ref[:, 3:4]
        ar = brow_ref[:, 5:6]
        cx1 = bt_ref[0:1, pl.ds(ibase, blk)]
        cy1 = bt_ref[1:2, pl.ds(ibase, blk)]
        cx2 = bt_ref[2:3, pl.ds(ibase, blk)]
        cy2 = bt_ref[3:4, pl.ds(ibase, blk)]
        ac = bt_ref[4:5, pl.ds(ibase, blk)]
        iw = jnp.maximum(jnp.minimum(x2, cx2) - jnp.maximum(x1, cx1), 0.0)
        ih = jnp.minimum(y2, cy2) - jnp.maximum(y1, cy1)
        overlap = 3.0 * (iw * ih) > ar + ac
        tri_r = lax.broadcasted_iota(jnp.int32, (blk, blk), 0)
        tri_c = lax.broadcasted_iota(jnp.int32, (blk, blk), 1)
        s_ref[...] = jnp.where((tri_c > tri_r) & overlap, 1.0, 0.0)

        keep0 = keep_ref[:, pl.ds(ibase, blk)]

        def _step(kb):
            cnt = jnp.dot(kb, s_ref[...],
                          preferred_element_type=jnp.float32)
            return keep0 * jnp.where(cnt > 0.5, 0.0, 1.0)

        def _cond(carry):
            return carry[1]

        def _sweep(carry):
            kb, _ = carry
            kb1 = _step(kb)
            kb2 = _step(kb1)
            changed = jnp.sum(jnp.abs(kb2 - kb1)) > 0.0
            return (kb2, changed)

        kb, _ = lax.while_loop(_cond, _sweep, (keep0, True))
        keep_ref[:, pl.ds(ibase, blk)] = kb
        kbc = kb.T
        kbc_ref[...] = kbc

        out_ref[...] = brow_ref[:, 0:5] * kbc

    @pl.when(j > i)
    def _tail():
        kbc = kbc_ref[...]
        x1 = jnp.where(kbc > 0.0, brow_ref[:, 0:1], 3.0e4)
        y1 = brow_ref[:, 1:2]
        x2 = brow_ref[:, 2:3]
        y2 = brow_ref[:, 3:4]
        ar = brow_ref[:, 5:6]
        cx1 = bt_ref[0:1, pl.ds(jbase, blk)]
        cy1 = bt_ref[1:2, pl.ds(jbase, blk)]
        cx2 = bt_ref[2:3, pl.ds(jbase, blk)]
        cy2 = bt_ref[3:4, pl.ds(jbase, blk)]
        ac = bt_ref[4:5, pl.ds(jbase, blk)]
        iw = jnp.maximum(jnp.minimum(x2, cx2) - jnp.maximum(x1, cx1), 0.0)
        ih = jnp.minimum(y2, cy2) - jnp.maximum(y1, cy1)
        overlap = 3.0 * (iw * ih) > ar + ac
        sup = jnp.any(overlap, axis=0, keepdims=True)
        keep_ref[:, pl.ds(jbase, blk)] = jnp.where(
            sup, 0.0, keep_ref[:, pl.ds(jbase, blk)])


def kernel(boxes, scores):
    n = boxes.shape[0]
    blk = _BLK
    nb = -(-n // blk)
    n_pad = nb * blk
    pad = n_pad - n

    neg, x1, y1, x2, y2, s = lax.sort(
        (-scores, boxes[:, 0], boxes[:, 1], boxes[:, 2], boxes[:, 3],
         scores),
        num_keys=1)
    area = (x2 - x1) * (y2 - y1)
    z = jnp.zeros((pad,), jnp.float32)
    x1 = jnp.concatenate([x1, z])
    y1 = jnp.concatenate([y1, z])
    x2 = jnp.concatenate([x2, z])
    y2 = jnp.concatenate([y2, z])
    s = jnp.concatenate([s, z])
    area = jnp.concatenate([area, z])
    bt = jnp.stack([x1, y1, x2, y2, area], axis=0)
    rows = jnp.stack([x1, y1, x2, y2, s, area], axis=1)

    out = pl.pallas_call(
        functools.partial(_nms_body, blk=blk),
        grid=(nb, nb),
        in_specs=[
            pl.BlockSpec((5, n_pad), lambda i, j: (0, 0)),
            pl.BlockSpec((blk, 6), lambda i, j: (i, 0)),
        ],
        out_specs=pl.BlockSpec((blk, 5), lambda i, j: (i, 0)),
        out_shape=jax.ShapeDtypeStruct((n_pad, 5), jnp.float32),
        scratch_shapes=[
            pltpu.VMEM((blk, blk), jnp.float32),
            pltpu.VMEM((1, n_pad), jnp.float32),
            pltpu.VMEM((blk, 1), jnp.float32),
        ],
    )(bt, rows)

    return out[:n]
